# Initial kernel scaffold; baseline (speedup 1.0000x reference)
import jax
import jax.numpy as jnp
from jax.experimental import pallas as pl


def kernel(x, last_update, edge_index, t, msg, params):
    raise NotImplementedError("write your pallas kernel here")


# speedup vs baseline: 11.4327x; 11.4327x over previous
import functools

import jax
import jax.numpy as jnp
from jax import lax
from jax.experimental import pallas as pl
from jax.experimental.pallas import tpu as pltpu
from jax.experimental.pallas import tpu_sc as plsc

F32 = jnp.float32

N = 10000
E = 320000
NC, NS = 2, 16
NW = NC * NS
PER_W = E // NW
CH = 80
NCHUNK = PER_W // CH
ROWS_T = 800
N_PAD = NS * ROWS_T
AW = 128
BN = 400
BE = 1000


def _sc_mesh():
    return plsc.VectorSubcoreMesh(core_axis_name="c", subcore_axis_name="s",
                                  num_cores=NC, num_subcores=NS)


def _gather_body(with_lu, q_hbm, kv_hbm, lu_hbm, dst_hbm, src_hbm, *rest):
    if with_lu:
        qd_hbm, kvs_hbm, lus_hbm, di, si, qbuf, kvbuf, lubuf = rest
    else:
        qd_hbm, kvs_hbm, di, si, qbuf, kvbuf = rest
    wid = lax.axis_index("s") * NC + lax.axis_index("c")
    base = wid * PER_W

    def step(j, carry):
        off = base + j * CH
        pltpu.sync_copy(dst_hbm.at[pl.ds(off, CH)], di)
        pltpu.sync_copy(src_hbm.at[pl.ds(off, CH)], si)
        pltpu.sync_copy(q_hbm.at[di], qbuf)
        pltpu.sync_copy(kv_hbm.at[si], kvbuf)
        pltpu.sync_copy(qbuf, qd_hbm.at[pl.ds(off, CH)])
        pltpu.sync_copy(kvbuf, kvs_hbm.at[pl.ds(off, CH)])
        if with_lu:
            pltpu.sync_copy(lu_hbm.at[si], lubuf)
            pltpu.sync_copy(lubuf, lus_hbm.at[pl.ds(off, CH)])
        return carry

    lax.fori_loop(0, NCHUNK, step, 0)


def _sc_gather(q, kv, lu, dst, src, with_lu):
    outs = [jax.ShapeDtypeStruct((E, 128), F32), jax.ShapeDtypeStruct((E, 256), F32)]
    scratch = [pltpu.VMEM((CH,), jnp.int32), pltpu.VMEM((CH,), jnp.int32),
               pltpu.VMEM((CH, 128), F32), pltpu.VMEM((CH, 256), F32)]
    if with_lu:
        outs.insert(2, jax.ShapeDtypeStruct((E,), F32))
        scratch.append(pltpu.VMEM((CH,), F32))
    fn = pl.kernel(functools.partial(_gather_body, with_lu),
                   out_type=tuple(outs), mesh=_sc_mesh(), scratch_types=scratch)
    return fn(q, kv, lu, dst, src)


def _scatter_body(wm_hbm, dst_hbm, z_hbm, parts_hbm, acc, buf, di):
    c = lax.axis_index("c")
    s = lax.axis_index("s")
    pltpu.sync_copy(z_hbm, buf)

    def zstep(k, carry):
        pltpu.sync_copy(buf, acc.at[pl.ds(s * ROWS_T + k * CH, CH)])
        return carry

    lax.fori_loop(0, ROWS_T // CH, zstep, 0)
    plsc.subcore_barrier()
    base = c * (E // NC) + s * PER_W

    def step(j, carry):
        off = base + j * CH
        pltpu.sync_copy(dst_hbm.at[pl.ds(off, CH)], di)
        pltpu.sync_copy(wm_hbm.at[pl.ds(off, CH)], buf)
        pltpu.sync_copy(buf, acc.at[di], add=True)
        return carry

    lax.fori_loop(0, NCHUNK, step, 0)
    plsc.subcore_barrier()
    row0 = c * N_PAD + s * ROWS_T

    def ostep(k, carry):
        pltpu.sync_copy(acc.at[pl.ds(s * ROWS_T + k * CH, CH)], buf)
        pltpu.sync_copy(buf, parts_hbm.at[pl.ds(row0 + k * CH, CH)])
        return carry

    lax.fori_loop(0, ROWS_T // CH, ostep, 0)


def _sc_scatter(wm, dst, zeros):
    fn = pl.kernel(_scatter_body,
                   out_type=jax.ShapeDtypeStruct((2 * N_PAD, AW), F32),
                   mesh=_sc_mesh(),
                   scratch_types=[pltpu.VMEM_SHARED((N_PAD, AW), F32),
                                  pltpu.VMEM((CH, AW), F32),
                                  pltpu.VMEM((CH,), jnp.int32)])
    return fn(wm, dst, zeros)


def _project_kernel(h_ref, w_ref, b_ref, q_ref, kv_ref, s_ref):
    out = jnp.dot(h_ref[...], w_ref[...], preferred_element_type=F32) + b_ref[...]
    q_ref[...] = out[:, :128]
    kv_ref[...] = out[:, 128:384]
    s_ref[...] = out[:, 384:]


def _tc_project(h, w_all, b_all):
    return pl.pallas_call(
        _project_kernel,
        grid=(N // BN,),
        in_specs=[pl.BlockSpec((BN, 128), lambda i: (i, 0)),
                  pl.BlockSpec((128, 512), lambda i: (0, 0)),
                  pl.BlockSpec((1, 512), lambda i: (0, 0))],
        out_specs=[pl.BlockSpec((BN, 128), lambda i: (i, 0)),
                   pl.BlockSpec((BN, 256), lambda i: (i, 0)),
                   pl.BlockSpec((BN, 128), lambda i: (i, 0))],
        out_shape=[jax.ShapeDtypeStruct((N, 128), F32),
                   jax.ShapeDtypeStruct((N, 256), F32),
                   jax.ShapeDtypeStruct((N, 128), F32)],
    )(h, w_all, b_all)


def _edge_kernel(qd_ref, kvs_ref, lus_ref, t_ref, msg_ref, tw_ref, tb_ref,
                 mw_ref, mb_ref, we_ref, be_ref, out0_ref, out1_ref):
    rel = lus_ref[...] - t_ref[...]
    enc = jnp.cos(rel * tw_ref[...] + tb_ref[...])
    mp = jnp.dot(msg_ref[...], mw_ref[...], preferred_element_type=F32) + mb_ref[...]
    ea = jnp.concatenate([enc, mp], axis=1)
    e = jnp.dot(ea, we_ref[...], preferred_element_type=F32) + be_ref[...]
    q = qd_ref[...]
    k = kvs_ref[:, :128] + e
    v = kvs_ref[:, 128:] + e
    a0 = jnp.sum(q[:, :64] * k[:, :64], axis=1, keepdims=True) * 0.125
    a1 = jnp.sum(q[:, 64:] * k[:, 64:], axis=1, keepdims=True) * 0.125
    ex0 = jnp.exp(a0)
    ex1 = jnp.exp(a1)
    pad = jnp.zeros((BE, 63), F32)
    out0_ref[...] = jnp.concatenate([v[:, :64] * ex0, ex0, pad], axis=1)
    out1_ref[...] = jnp.concatenate([v[:, 64:] * ex1, ex1, pad], axis=1)


def _tc_edge(qd, kvs, lus, t2, msg, tw, tb, mw, mb, we, be):
    return pl.pallas_call(
        _edge_kernel,
        grid=(E // BE,),
        in_specs=[pl.BlockSpec((BE, 128), lambda i: (i, 0)),
                  pl.BlockSpec((BE, 256), lambda i: (i, 0)),
                  pl.BlockSpec((BE, 1), lambda i: (i, 0)),
                  pl.BlockSpec((BE, 1), lambda i: (i, 0)),
                  pl.BlockSpec((BE, 16), lambda i: (i, 0)),
                  pl.BlockSpec((1, 64), lambda i: (0, 0)),
                  pl.BlockSpec((1, 64), lambda i: (0, 0)),
                  pl.BlockSpec((16, 64), lambda i: (0, 0)),
                  pl.BlockSpec((1, 64), lambda i: (0, 0)),
                  pl.BlockSpec((128, 128), lambda i: (0, 0)),
                  pl.BlockSpec((1, 128), lambda i: (0, 0))],
        out_specs=[pl.BlockSpec((BE, AW), lambda i: (i, 0)),
                   pl.BlockSpec((BE, AW), lambda i: (i, 0))],
        out_shape=[jax.ShapeDtypeStruct((E, AW), F32),
                   jax.ShapeDtypeStruct((E, AW), F32)],
    )(qd, kvs, lus, t2, msg, tw, tb, mw, mb, we, be)


def _combine(a0p0, a0p1, a1p0, a1p1, s_prev, g, b):
    agg0 = a0p0 + a0p1
    agg1 = a1p0 + a1p1
    den0 = agg0[:, 64:65] + 1e-16
    den1 = agg1[:, 64:65] + 1e-16
    o = jnp.concatenate([agg0[:, :64] / den0, agg1[:, :64] / den1], axis=1)
    o = o + s_prev
    mu = jnp.mean(o, axis=1, keepdims=True)
    var = jnp.mean((o - mu) ** 2, axis=1, keepdims=True)
    return (o - mu) / jnp.sqrt(var + 1e-5) * g + b


def _mid_kernel(a0p0_ref, a0p1_ref, a1p0_ref, a1p1_ref, sp_ref, g_ref, b_ref,
                w_ref, ba_ref, q_ref, kv_ref, s_ref):
    h = _combine(a0p0_ref[...], a0p1_ref[...], a1p0_ref[...], a1p1_ref[...],
                 sp_ref[...], g_ref[...], b_ref[...])
    h = jnp.maximum(h, 0.0)
    out = jnp.dot(h, w_ref[...], preferred_element_type=F32) + ba_ref[...]
    q_ref[...] = out[:, :128]
    kv_ref[...] = out[:, 128:384]
    s_ref[...] = out[:, 384:]


_PARTS_SPECS = [pl.BlockSpec((BN, AW), lambda i: (i, 0)),
                pl.BlockSpec((BN, AW), lambda i: (i + N_PAD // BN, 0)),
                pl.BlockSpec((BN, AW), lambda i: (i, 0)),
                pl.BlockSpec((BN, AW), lambda i: (i + N_PAD // BN, 0))]


def _tc_mid(parts0, parts1, s_prev, g, b, w_all, b_all):
    return pl.pallas_call(
        _mid_kernel,
        grid=(N // BN,),
        in_specs=_PARTS_SPECS +
                 [pl.BlockSpec((BN, 128), lambda i: (i, 0)),
                  pl.BlockSpec((1, 128), lambda i: (0, 0)),
                  pl.BlockSpec((1, 128), lambda i: (0, 0)),
                  pl.BlockSpec((128, 512), lambda i: (0, 0)),
                  pl.BlockSpec((1, 512), lambda i: (0, 0))],
        out_specs=[pl.BlockSpec((BN, 128), lambda i: (i, 0)),
                   pl.BlockSpec((BN, 256), lambda i: (i, 0)),
                   pl.BlockSpec((BN, 128), lambda i: (i, 0))],
        out_shape=[jax.ShapeDtypeStruct((N, 128), F32),
                   jax.ShapeDtypeStruct((N, 256), F32),
                   jax.ShapeDtypeStruct((N, 128), F32)],
    )(parts0, parts0, parts1, parts1, s_prev, g, b, w_all, b_all)


def _post_kernel(a0p0_ref, a0p1_ref, a1p0_ref, a1p1_ref, sp_ref, g_ref, b_ref,
                 h_ref):
    h_ref[...] = _combine(a0p0_ref[...], a0p1_ref[...], a1p0_ref[...],
                          a1p1_ref[...], sp_ref[...], g_ref[...], b_ref[...])


def _tc_post(parts0, parts1, s_prev, g, b):
    return pl.pallas_call(
        _post_kernel,
        grid=(N // BN,),
        in_specs=_PARTS_SPECS +
                 [pl.BlockSpec((BN, 128), lambda i: (i, 0)),
                  pl.BlockSpec((1, 128), lambda i: (0, 0)),
                  pl.BlockSpec((1, 128), lambda i: (0, 0))],
        out_specs=pl.BlockSpec((BN, 128), lambda i: (i, 0)),
        out_shape=jax.ShapeDtypeStruct((N, 128), F32),
    )(parts0, parts0, parts1, parts1, s_prev, g, b)


def kernel(x, last_update, edge_index, t, msg, params):
    p = params
    src = edge_index[0]
    dst = edge_index[1]
    t2 = t.reshape(E, 1)
    zeros = jnp.zeros((CH, AW), F32)
    tw = p['time_w']
    tb = p['time_b'].reshape(1, 64)
    mw = p['msg_w']
    mb = p['msg_b'].reshape(1, 64)
    w_all, b_all, ew, eb, lng, lnb = [], [], [], [], [], []
    for i in range(3):
        w_all.append(jnp.concatenate([p['c%d_q_w' % i], p['c%d_k_w' % i],
                                      p['c%d_v_w' % i], p['c%d_s_w' % i]], axis=1))
        b_all.append(jnp.concatenate([p['c%d_q_b' % i], p['c%d_k_b' % i],
                                      p['c%d_v_b' % i], p['c%d_s_b' % i]]).reshape(1, 512))
        ew.append(p['c%d_e_w' % i])
        eb.append(p['c%d_e_b' % i].reshape(1, 128))
        lng.append(p['ln%d_g' % i].reshape(1, 128))
        lnb.append(p['ln%d_b' % i].reshape(1, 128))

    q, kv, s = _tc_project(x, w_all[0], b_all[0])
    qd, kvs, lus = _sc_gather(q, kv, last_update, dst, src, with_lu=True)
    lus2 = lus.reshape(E, 1)

    wm0, wm1 = _tc_edge(qd, kvs, lus2, t2, msg, tw, tb, mw, mb, ew[0], eb[0])
    parts0 = _sc_scatter(wm0, dst, zeros)
    parts1 = _sc_scatter(wm1, dst, zeros)
    q, kv, s = _tc_mid(parts0, parts1, s, lng[0], lnb[0], w_all[1], b_all[1])

    qd, kvs = _sc_gather(q, kv, last_update, dst, src, with_lu=False)
    wm0, wm1 = _tc_edge(qd, kvs, lus2, t2, msg, tw, tb, mw, mb, ew[1], eb[1])
    parts0 = _sc_scatter(wm0, dst, zeros)
    parts1 = _sc_scatter(wm1, dst, zeros)
    q, kv, s = _tc_mid(parts0, parts1, s, lng[1], lnb[1], w_all[2], b_all[2])

    qd, kvs = _sc_gather(q, kv, last_update, dst, src, with_lu=False)
    wm0, wm1 = _tc_edge(qd, kvs, lus2, t2, msg, tw, tb, mw, mb, ew[2], eb[2])
    parts0 = _sc_scatter(wm0, dst, zeros)
    parts1 = _sc_scatter(wm1, dst, zeros)
    return _tc_post(parts0, parts1, s, lng[2], lnb[2])
